# baseline (speedup 1.0000x reference)
import functools

import jax
import jax.numpy as jnp
from jax import lax
from jax.experimental import pallas as pl
from jax.experimental.pallas import tpu as pltpu
from jax.experimental.pallas import tpu_sc as plsc

CROP_RATIO = 0.8


@functools.lru_cache(maxsize=None)
def _crop_call(B, L, C, crop_len):
    info = plsc.get_sparse_core_info()
    NC, NS, NL = info.num_cores, info.num_subcores, info.num_lanes
    NW = NC * NS
    assert B == NW, "one subcore per batch element"
    assert B % NL == 0

    NBUF = 4
    max_rows = (131071 - B - 1024) // (NBUF * C)
    nch = -(-crop_len // max_rows)
    chunk = -(-crop_len // nch)
    sizes = []
    off = 0
    offs = []
    while off < crop_len:
        sz = min(chunk, crop_len - off)
        offs.append(off)
        sizes.append(sz)
        off += sz
    nch = len(sizes)

    mesh = plsc.VectorSubcoreMesh(core_axis_name="c", subcore_axis_name="s")

    @functools.partial(
        pl.kernel,
        mesh=mesh,
        compiler_params=pltpu.CompilerParams(
            use_tc_tiling_on_sc=False, needs_layout_passes=False
        ),
        out_type=jax.ShapeDtypeStruct((B, crop_len, C), jnp.float32),
        scratch_types=[
            pltpu.VMEM((B,), jnp.int32),
        ]
        + [pltpu.VMEM((chunk, C), jnp.float32) for _ in range(NBUF)]
        + [pltpu.SemaphoreType.DMA for _ in range(2 * NBUF)],
    )
    def k(x_hbm, start_hbm, out_hbm, start_v, *bufs_sems):
        bufs = bufs_sems[:NBUF]
        rsems = bufs_sems[NBUF : 2 * NBUF]
        wsems = bufs_sems[2 * NBUF :]
        wid = lax.axis_index("c") * NS + lax.axis_index("s")
        pltpu.sync_copy(start_hbm, start_v)

        lane = lax.iota(jnp.int32, NL)
        group = jnp.where(wid < NL, start_v[pl.ds(0, NL)], start_v[pl.ds(NL, NL)])
        s = jnp.max(jnp.where(lane == wid % NL, group, 0))

        def rd(i):
            return pltpu.make_async_copy(
                x_hbm.at[wid, pl.ds(s + offs[i], sizes[i]), :],
                bufs[i % NBUF].at[pl.ds(0, sizes[i]), :],
                rsems[i % NBUF],
            )

        def wr(i):
            return pltpu.make_async_copy(
                bufs[i % NBUF].at[pl.ds(0, sizes[i]), :],
                out_hbm.at[wid, pl.ds(offs[i], sizes[i]), :],
                wsems[i % NBUF],
            )

        ahead = NBUF // 2
        for i in range(min(ahead, nch)):
            rd(i).start()
        for i in range(nch):
            rd(i).wait()
            wr(i).start()
            j = i + ahead
            if j < nch:
                if j - NBUF >= 0:
                    wr(j - NBUF).wait()
                rd(j).start()
        for i in range(max(0, nch - NBUF), nch):
            wr(i).wait()

    return k


def kernel(x):
    B, L, C = x.shape
    crop_len = int(L * CROP_RATIO)
    start = jax.random.randint(
        jax.random.key(1), (B,), 0, L - crop_len + 1
    ).astype(jnp.int32)
    return _crop_call(B, L, C, crop_len)(x, start)


# speedup vs baseline: 1.8383x; 1.8383x over previous
import functools

import jax
import jax.numpy as jnp
from jax import lax
from jax.experimental import pallas as pl
from jax.experimental.pallas import tpu as pltpu
from jax.experimental.pallas import tpu_sc as plsc

CROP_RATIO = 0.8


@functools.lru_cache(maxsize=None)
def _crop_call(B, L, C, crop_len):
    info = plsc.get_sparse_core_info()
    NC, NS, NL = info.num_cores, info.num_subcores, info.num_lanes
    NW = NC * NS
    assert B == NW, "one subcore per batch element"
    assert B % NL == 0

    CHUNK = 128
    nch = crop_len // CHUNK
    rem = crop_len % CHUNK
    rem_pad = -(-rem // 8) * 8
    IDX = -(-max(crop_len, nch * CHUNK + rem_pad) // NL) * NL
    NBUF = 3

    mesh = plsc.VectorSubcoreMesh(core_axis_name="c", subcore_axis_name="s")

    @functools.partial(
        pl.kernel,
        mesh=mesh,
        compiler_params=pltpu.CompilerParams(needs_layout_passes=False),
        out_type=jax.ShapeDtypeStruct((B, crop_len, C), jnp.float32),
        scratch_types=[
            pltpu.VMEM((B,), jnp.int32),
            pltpu.VMEM((IDX,), jnp.int32),
        ]
        + [pltpu.VMEM((CHUNK, C), jnp.float32) for _ in range(NBUF)]
        + ([pltpu.VMEM((rem_pad, C), jnp.float32)] if rem else [])
        + [pltpu.SemaphoreType.DMA for _ in range(2 * NBUF + 2)],
    )
    def k(x_hbm, start_hbm, out_hbm, start_v, idx_v, *bufs_sems):
        bufs = bufs_sems[:NBUF]
        nb = NBUF + (1 if rem else 0)
        buf_last = bufs_sems[NBUF] if rem else None
        rsems = bufs_sems[nb : nb + NBUF + 1]
        wsems = bufs_sems[nb + NBUF + 1 :]
        wid = lax.axis_index("c") * NS + lax.axis_index("s")
        pltpu.sync_copy(start_hbm, start_v)

        lane = lax.iota(jnp.int32, NL)
        group = jnp.where(wid < NL, start_v[pl.ds(0, NL)], start_v[pl.ds(NL, NL)])
        s_splat = jnp.take(group, jnp.full((NL,), wid % NL, jnp.int32))

        base = s_splat + lane
        for j in range(IDX // NL):
            idx_v[pl.ds(NL * j, NL)] = jnp.minimum(
                base + NL * j, s_splat + crop_len - 1
            )

        def rd(i):
            if i < nch:
                return pltpu.make_async_copy(
                    x_hbm.at[wid].at[idx_v.at[pl.ds(i * CHUNK, CHUNK)]],
                    bufs[i % NBUF],
                    rsems[i % NBUF],
                )
            return pltpu.make_async_copy(
                x_hbm.at[wid].at[idx_v.at[pl.ds(nch * CHUNK, rem_pad)]],
                buf_last,
                rsems[NBUF],
            )

        def wr(i):
            if i < nch:
                return pltpu.make_async_copy(
                    bufs[i % NBUF],
                    out_hbm.at[wid, pl.ds(i * CHUNK, CHUNK), :],
                    wsems[i % NBUF],
                )
            r8 = rem - rem % 8
            cps = []
            if r8:
                cps.append(
                    pltpu.make_async_copy(
                        buf_last.at[pl.ds(0, r8), :],
                        out_hbm.at[wid, pl.ds(nch * CHUNK, r8), :],
                        wsems[NBUF],
                    )
                )
            for c0 in range(0, C, 128):
                cps.append(
                    pltpu.make_async_copy(
                        buf_last.at[pl.ds(r8, rem - r8), pl.ds(c0, 128)],
                        out_hbm.at[
                            wid, pl.ds(nch * CHUNK + r8, rem - r8), pl.ds(c0, 128)
                        ],
                        wsems[NBUF],
                    )
                )
            return cps

        if rem:
            rd(nch).start()

        ahead = NBUF - 1
        for i in range(min(ahead, nch)):
            rd(i).start()
        for i in range(nch):
            rd(i).wait()
            wr(i).start()
            j = i + ahead
            if j < nch:
                if j - NBUF >= 0:
                    wr(j - NBUF).wait()
                rd(j).start()
        if rem:
            rd(nch).wait()
            for cp in wr(nch):
                cp.start()
        for i in range(max(0, nch - NBUF), nch):
            wr(i).wait()
        if rem:
            for cp in wr(nch):
                cp.wait()

    return k


def kernel(x):
    B, L, C = x.shape
    crop_len = int(L * CROP_RATIO)
    start = jax.random.randint(
        jax.random.key(1), (B,), 0, L - crop_len + 1
    ).astype(jnp.int32)
    return _crop_call(B, L, C, crop_len)(x, start)
